# baseline (speedup 1.0000x reference)
import functools

import jax
import jax.numpy as jnp
from jax import lax
from jax.experimental import pallas as pl
from jax.experimental.pallas import tpu as pltpu
from jax.experimental.pallas import tpu_sc as plsc

E = 8
K = 2
D = 1024
F = 4096
CAP_FACTOR = 1.25

N = 4096
C = int(CAP_FACTOR * N * K / E)
NROWS = (E + 1) * C
DUMMY = E * C

TB = 512
NB = N // TB

NW = 32
TPW = N // NW
CH = 32
CHC = 16
NCHC = TPW // CHC
D2 = D // 2

BC = 128
NCB = C // BC
BF = 1024
NFB = F // BF


def _rne_bf16_bits(v):
    b = lax.bitcast_convert_type(v, jnp.int32)
    return b + 0x7FFF + ((b >> 16) & 1)


def _router_body(x_ref, wr_ref, s0_ref, s1_ref, w0_ref, w1_ref, aux_ref,
                 cnt_ref, xpk_ref, scr):
    i = pl.program_id(0)

    @pl.when(i == 0)
    def _():
        scr[...] = jnp.zeros_like(scr)

    lo = (_rne_bf16_bits(x_ref[:, 0:D2]) >> 16) & 0xFFFF
    hi = _rne_bf16_bits(x_ref[:, D2:D]) & ~0xFFFF
    xpk_ref[...] = lo | hi

    logits = jnp.dot(x_ref[...], wr_ref[...], preferred_element_type=jnp.float32)
    m = jnp.max(logits, axis=-1, keepdims=True)
    ex = jnp.exp(logits - m)
    probs = ex / jnp.sum(ex, axis=-1, keepdims=True)

    lane = lax.broadcasted_iota(jnp.int32, (TB, E), 1)
    m0 = jnp.max(probs, axis=-1, keepdims=True)
    e0 = jnp.min(jnp.where(probs == m0, lane, E), axis=-1, keepdims=True)
    oh0 = lane == e0
    pr2 = jnp.where(oh0, -jnp.inf, probs)
    m1 = jnp.max(pr2, axis=-1, keepdims=True)
    e1 = jnp.min(jnp.where(pr2 == m1, lane, E), axis=-1, keepdims=True)
    oh1 = lane == e1

    ssum = m0 + m1
    g0 = m0 / ssum
    g1 = m1 / ssum

    oh0f = oh0.astype(jnp.float32)
    oh1f = oh1.astype(jnp.float32)
    ohtf = oh0f + oh1f

    ri = lax.broadcasted_iota(jnp.int32, (TB, TB), 0)
    cj = lax.broadcasted_iota(jnp.int32, (TB, TB), 1)
    lstrict = (cj < ri).astype(jnp.float32)
    cume = jnp.dot(lstrict, ohtf, preferred_element_type=jnp.float32)

    carry = scr[0:1, 0:E]
    cnt_excl = cume + carry
    pos0 = jnp.sum(oh0f * cnt_excl, axis=-1, keepdims=True)
    pos1 = jnp.sum(oh1f * cnt_excl, axis=-1, keepdims=True)

    cf = jnp.float32(C)
    keep0 = pos0 < cf
    keep1 = pos1 < cf
    e0f = e0.astype(jnp.float32)
    e1f = e1.astype(jnp.float32)
    slot0 = jnp.where(keep0, e0f * cf + jnp.minimum(pos0, cf - 1), jnp.float32(DUMMY))
    slot1 = jnp.where(keep1, e1f * cf + jnp.minimum(pos1, cf - 1), jnp.float32(DUMMY))

    s0_ref[...] = slot0.astype(jnp.int32)
    s1_ref[...] = slot1.astype(jnp.int32)
    w0_ref[...] = jnp.broadcast_to(jnp.where(keep0, g0, 0.0), (TB, 16))
    w1_ref[...] = jnp.broadcast_to(jnp.where(keep1, g1, 0.0), (TB, 16))

    carry_new = carry + jnp.sum(ohtf, axis=0, keepdims=True)
    me_new = scr[1:2, 0:E] + jnp.sum(probs, axis=0, keepdims=True)
    scr[0:1, 0:E] = carry_new
    scr[1:2, 0:E] = me_new

    @pl.when(i == NB - 1)
    def _():
        ce = carry_new / jnp.float32(K * N)
        me = me_new / jnp.float32(N)
        aux = jnp.float32(E) * jnp.sum(me * ce)
        aux_ref[...] = jnp.full((8, 128), aux, jnp.float32)
        cnt_ref[...] = jnp.broadcast_to(carry_new, (8, E)).astype(jnp.int32)


def _router(x, wr):
    outs = pl.pallas_call(
        _router_body,
        grid=(NB,),
        in_specs=[
            pl.BlockSpec((TB, D), lambda i: (i, 0)),
            pl.BlockSpec((D, E), lambda i: (0, 0)),
        ],
        out_specs=[
            pl.BlockSpec((TB, 1), lambda i: (i, 0)),
            pl.BlockSpec((TB, 1), lambda i: (i, 0)),
            pl.BlockSpec((TB, 16), lambda i: (i, 0)),
            pl.BlockSpec((TB, 16), lambda i: (i, 0)),
            pl.BlockSpec((8, 128), lambda i: (0, 0)),
            pl.BlockSpec((8, E), lambda i: (0, 0)),
            pl.BlockSpec((TB, D2), lambda i: (i, 0)),
        ],
        out_shape=[
            jax.ShapeDtypeStruct((N, 1), jnp.int32),
            jax.ShapeDtypeStruct((N, 1), jnp.int32),
            jax.ShapeDtypeStruct((N, 16), jnp.float32),
            jax.ShapeDtypeStruct((N, 16), jnp.float32),
            jax.ShapeDtypeStruct((8, 128), jnp.float32),
            jax.ShapeDtypeStruct((8, E), jnp.int32),
            jax.ShapeDtypeStruct((N, D2), jnp.int32),
        ],
        scratch_shapes=[pltpu.VMEM((8, 128), jnp.float32)],
    )(x, wr)
    s0, s1, w0, w1, auxb, cnt, xpk = outs
    return s0.reshape(N), s1.reshape(N), w0, w1, auxb[0, 0], cnt[0], xpk


def _dispatch(x, slot0, slot1):
    mesh = plsc.VectorSubcoreMesh(core_axis_name="c", subcore_axis_name="s")

    @functools.partial(
        pl.kernel,
        out_type=jax.ShapeDtypeStruct((NROWS, D2), jnp.int32),
        mesh=mesh,
        scratch_types=[
            pltpu.VMEM((CH,), jnp.int32),
            pltpu.VMEM((CH,), jnp.int32),
            pltpu.VMEM((CH, D2), jnp.int32),
            pltpu.SemaphoreType.DMA,
            pltpu.SemaphoreType.DMA,
        ],
    )
    def k(x_hbm, s0_hbm, s1_hbm, buf_hbm, i0, i1, rows, sem0, sem1):
        wid = lax.axis_index("s") * 2 + lax.axis_index("c")

        def chunk(c, acc):
            base = wid * TPW + c * CH
            pltpu.sync_copy(s0_hbm.at[pl.ds(base, CH)], i0)
            pltpu.sync_copy(s1_hbm.at[pl.ds(base, CH)], i1)
            pltpu.sync_copy(x_hbm.at[pl.ds(base, CH)], rows)
            c0 = pltpu.async_copy(rows, buf_hbm.at[i0], sem0)
            c1 = pltpu.async_copy(rows, buf_hbm.at[i1], sem1)
            c0.wait()
            c1.wait()
            return acc

        lax.fori_loop(0, TPW // CH, chunk, 0)

    return k(x, slot0, slot1)


def _ffn_body(cnt_ref, buf_ref, w1_ref, b1_ref, w2_ref, b2_ref, y_ref):
    e = pl.program_id(0)
    f = pl.program_id(1)

    @pl.when(jnp.logical_and(e == E, f == 0))
    def _():
        y_ref[...] = jnp.zeros_like(y_ref)

    @pl.when(e < E)
    def _():
        cnt = cnt_ref[jnp.minimum(e, E - 1)]
        for i in range(NCB):
            @pl.when(i * BC < cnt)
            def _():
                rows = pl.ds(i * BC, BC)
                bi = buf_ref[rows, :]
                xl = lax.bitcast_convert_type(bi << 16, jnp.float32)
                xh = lax.bitcast_convert_type(bi & ~0xFFFF, jnp.float32)
                xb = jnp.concatenate([xl, xh], axis=1)
                h = jnp.dot(xb, w1_ref[0],
                            preferred_element_type=jnp.float32)
                h = jnp.maximum(h + b1_ref[0], 0.0)
                part = jnp.dot(h, w2_ref[0], preferred_element_type=jnp.float32)

                @pl.when(f == 0)
                def _():
                    y_ref[rows, :] = part + b2_ref[0]

                @pl.when(f > 0)
                def _():
                    y_ref[rows, :] = y_ref[rows, :] + part


def _ffn(cnt, buf, w1r, b1r, w2r, b2r):
    emax = E - 1
    return pl.pallas_call(
        _ffn_body,
        grid=(E + 1, NFB),
        in_specs=[
            pl.BlockSpec(memory_space=pltpu.SMEM),
            pl.BlockSpec((C, D2), lambda e, f: (e, 0)),
            pl.BlockSpec((1, D, BF), lambda e, f: (jnp.minimum(e, emax), 0, f)),
            pl.BlockSpec((1, 1, BF), lambda e, f: (jnp.minimum(e, emax), 0, f)),
            pl.BlockSpec((1, BF, D), lambda e, f: (jnp.minimum(e, emax), f, 0)),
            pl.BlockSpec((1, 1, D), lambda e, f: (jnp.minimum(e, emax), 0, 0)),
        ],
        out_specs=pl.BlockSpec((C, D), lambda e, f: (e, 0)),
        out_shape=jax.ShapeDtypeStruct((NROWS, D), jnp.float32),
    )(cnt, buf, w1r, b1r, w2r, b2r)


def _combine(y, slot0, slot1, w0, w1):
    mesh = plsc.VectorSubcoreMesh(core_axis_name="c", subcore_axis_name="s")
    s0r = slot0.reshape(NW, NCHC, CHC)
    s1r = slot1.reshape(NW, NCHC, CHC)

    @functools.partial(
        pl.kernel,
        out_type=jax.ShapeDtypeStruct((N, D), jnp.float32),
        mesh=mesh,
        scratch_types=[
            pltpu.VMEM((NCHC, CHC), jnp.int32),
            pltpu.VMEM((NCHC, CHC), jnp.int32),
            pltpu.VMEM((TPW, 16), jnp.float32),
            pltpu.VMEM((TPW, 16), jnp.float32),
            pltpu.VMEM((CHC, D), jnp.float32),
            pltpu.VMEM((CHC, D), jnp.float32),
            pltpu.VMEM((CHC, D), jnp.float32),
            pltpu.VMEM((CHC, D), jnp.float32),
            pltpu.SemaphoreType.DMA,
            pltpu.SemaphoreType.DMA,
            pltpu.SemaphoreType.DMA,
            pltpu.SemaphoreType.DMA,
        ],
    )
    def k(y_hbm, s0_hbm, s1_hbm, w0_hbm, w1_hbm, out_hbm,
          i0, i1, wv0, wv1, r0a, r1a, r0b, r1b, sa0, sa1, sb0, sb1):
        wid = lax.axis_index("s") * 2 + lax.axis_index("c")
        base = wid * TPW
        pltpu.sync_copy(s0_hbm.at[wid], i0)
        pltpu.sync_copy(s1_hbm.at[wid], i1)
        pltpu.sync_copy(w0_hbm.at[pl.ds(base, TPW)], wv0)
        pltpu.sync_copy(w1_hbm.at[pl.ds(base, TPW)], wv1)

        bufs = ((r0a, r1a, sa0, sa1), (r0b, r1b, sb0, sb1))

        def issue(c, bs):
            g0 = pltpu.async_copy(y_hbm.at[i0.at[c]], bs[0], bs[2])
            g1 = pltpu.async_copy(y_hbm.at[i1.at[c]], bs[1], bs[3])
            return g0, g1

        pend = issue(0, bufs[0])
        for c in range(NCHC):
            r0, r1 = bufs[c % 2][0], bufs[c % 2][1]
            pend[0].wait()
            pend[1].wait()
            if c + 1 < NCHC:
                pend = issue(c + 1, bufs[(c + 1) % 2])

            def tok(t, a2):
                b0 = wv0[c * CHC + t, :]
                b1v = wv1[c * CHC + t, :]

                def dslice(j, a3):
                    for u in range(4):
                        sl = pl.ds((j * 4 + u) * 16, 16)
                        r0[t, sl] = b0 * r0[t, sl] + b1v * r1[t, sl]
                    return a3

                return lax.fori_loop(0, D // 64, dslice, a2)

            lax.fori_loop(0, CHC, tok, 0)
            pltpu.sync_copy(r0, out_hbm.at[pl.ds(base + c * CHC, CHC)])

    return k(y, s0r, s1r, w0, w1)


def kernel(hidden_states, Wr, W1, b1, W2, b2):
    orig_shape = hidden_states.shape
    x = hidden_states.reshape(-1, D)

    slot0, slot1, w0, w1, aux, cnt, xpk = _router(x, Wr)
    buf = _dispatch(xpk, slot0, slot1)
    y = _ffn(cnt, buf, W1, b1.reshape(E, 1, F), W2, b2.reshape(E, 1, D))
    out = _combine(y, slot0, slot1, w0, w1)
    return out.reshape(orig_shape), aux


# speedup vs baseline: 1.1692x; 1.1692x over previous
import functools

import jax
import jax.numpy as jnp
from jax import lax
from jax.experimental import pallas as pl
from jax.experimental.pallas import tpu as pltpu
from jax.experimental.pallas import tpu_sc as plsc

E = 8
K = 2
D = 1024
F = 4096
CAP_FACTOR = 1.25

N = 4096
C = int(CAP_FACTOR * N * K / E)
NROWS = (E + 1) * C
DUMMY = E * C

TB = 512
NB = N // TB

NW = 32
TPW = N // NW
CH = 32
CHC = 16
NCHC = TPW // CHC
D2 = D // 2

BC = 256
NCB = C // BC
BF = 1024
NFB = F // BF


def _rne_bf16_bits(v):
    b = lax.bitcast_convert_type(v, jnp.int32)
    return b + 0x7FFF + ((b >> 16) & 1)


def _router_body(x_ref, wr_ref, s0_ref, s1_ref, w0_ref, w1_ref, aux_ref,
                 cnt_ref, xpk_ref, scr):
    i = pl.program_id(0)

    @pl.when(i == 0)
    def _():
        scr[...] = jnp.zeros_like(scr)

    lo = (_rne_bf16_bits(x_ref[:, 0:D2]) >> 16) & 0xFFFF
    hi = _rne_bf16_bits(x_ref[:, D2:D]) & ~0xFFFF
    xpk_ref[...] = lo | hi

    logits = jnp.dot(x_ref[...], wr_ref[...], preferred_element_type=jnp.float32)
    m = jnp.max(logits, axis=-1, keepdims=True)
    ex = jnp.exp(logits - m)
    probs = ex / jnp.sum(ex, axis=-1, keepdims=True)

    lane = lax.broadcasted_iota(jnp.int32, (TB, E), 1)
    m0 = jnp.max(probs, axis=-1, keepdims=True)
    e0 = jnp.min(jnp.where(probs == m0, lane, E), axis=-1, keepdims=True)
    oh0 = lane == e0
    pr2 = jnp.where(oh0, -jnp.inf, probs)
    m1 = jnp.max(pr2, axis=-1, keepdims=True)
    e1 = jnp.min(jnp.where(pr2 == m1, lane, E), axis=-1, keepdims=True)
    oh1 = lane == e1

    ssum = m0 + m1
    g0 = m0 / ssum
    g1 = m1 / ssum

    oh0f = oh0.astype(jnp.float32)
    oh1f = oh1.astype(jnp.float32)
    ohtf = oh0f + oh1f

    ri = lax.broadcasted_iota(jnp.int32, (TB, TB), 0)
    cj = lax.broadcasted_iota(jnp.int32, (TB, TB), 1)
    lstrict = (cj < ri).astype(jnp.float32)
    cume = jnp.dot(lstrict, ohtf, preferred_element_type=jnp.float32)

    carry = scr[0:1, 0:E]
    cnt_excl = cume + carry
    pos0 = jnp.sum(oh0f * cnt_excl, axis=-1, keepdims=True)
    pos1 = jnp.sum(oh1f * cnt_excl, axis=-1, keepdims=True)

    cf = jnp.float32(C)
    keep0 = pos0 < cf
    keep1 = pos1 < cf
    e0f = e0.astype(jnp.float32)
    e1f = e1.astype(jnp.float32)
    slot0 = jnp.where(keep0, e0f * cf + jnp.minimum(pos0, cf - 1), jnp.float32(DUMMY))
    slot1 = jnp.where(keep1, e1f * cf + jnp.minimum(pos1, cf - 1), jnp.float32(DUMMY))

    s0_ref[...] = slot0.astype(jnp.int32)
    s1_ref[...] = slot1.astype(jnp.int32)
    w0_ref[...] = jnp.broadcast_to(jnp.where(keep0, g0, 0.0), (TB, 16))
    w1_ref[...] = jnp.broadcast_to(jnp.where(keep1, g1, 0.0), (TB, 16))

    carry_new = carry + jnp.sum(ohtf, axis=0, keepdims=True)
    me_new = scr[1:2, 0:E] + jnp.sum(probs, axis=0, keepdims=True)
    scr[0:1, 0:E] = carry_new
    scr[1:2, 0:E] = me_new

    @pl.when(i == NB - 1)
    def _():
        ce = carry_new / jnp.float32(K * N)
        me = me_new / jnp.float32(N)
        aux = jnp.float32(E) * jnp.sum(me * ce)
        aux_ref[...] = jnp.full((8, 128), aux, jnp.float32)
        cnt_ref[...] = jnp.broadcast_to(carry_new, (8, E)).astype(jnp.int32)


def _router(x, wr):
    outs = pl.pallas_call(
        _router_body,
        grid=(NB,),
        in_specs=[
            pl.BlockSpec((TB, D), lambda i: (i, 0)),
            pl.BlockSpec((D, E), lambda i: (0, 0)),
        ],
        out_specs=[
            pl.BlockSpec((TB, 1), lambda i: (i, 0)),
            pl.BlockSpec((TB, 1), lambda i: (i, 0)),
            pl.BlockSpec((TB, 16), lambda i: (i, 0)),
            pl.BlockSpec((TB, 16), lambda i: (i, 0)),
            pl.BlockSpec((8, 128), lambda i: (0, 0)),
            pl.BlockSpec((8, E), lambda i: (0, 0)),
            pl.BlockSpec((TB, D2), lambda i: (i, 0)),
        ],
        out_shape=[
            jax.ShapeDtypeStruct((N, 1), jnp.int32),
            jax.ShapeDtypeStruct((N, 1), jnp.int32),
            jax.ShapeDtypeStruct((N, 16), jnp.float32),
            jax.ShapeDtypeStruct((N, 16), jnp.float32),
            jax.ShapeDtypeStruct((8, 128), jnp.float32),
            jax.ShapeDtypeStruct((8, E), jnp.int32),
            jax.ShapeDtypeStruct((N, D2), jnp.int32),
        ],
        scratch_shapes=[pltpu.VMEM((8, 128), jnp.float32)],
    )(x, wr)
    s0, s1, w0, w1, auxb, cnt, xpk = outs
    return s0.reshape(N), s1.reshape(N), w0, w1, auxb[0, 0], cnt[0], xpk


def _dispatch(x, slot0, slot1):
    mesh = plsc.VectorSubcoreMesh(core_axis_name="c", subcore_axis_name="s")

    @functools.partial(
        pl.kernel,
        out_type=jax.ShapeDtypeStruct((NROWS, D2), jnp.int32),
        mesh=mesh,
        scratch_types=[
            pltpu.VMEM((CH,), jnp.int32),
            pltpu.VMEM((CH,), jnp.int32),
            pltpu.VMEM((CH, D2), jnp.int32),
            pltpu.SemaphoreType.DMA,
            pltpu.SemaphoreType.DMA,
        ],
    )
    def k(x_hbm, s0_hbm, s1_hbm, buf_hbm, i0, i1, rows, sem0, sem1):
        wid = lax.axis_index("s") * 2 + lax.axis_index("c")

        def chunk(c, acc):
            base = wid * TPW + c * CH
            pltpu.sync_copy(s0_hbm.at[pl.ds(base, CH)], i0)
            pltpu.sync_copy(s1_hbm.at[pl.ds(base, CH)], i1)
            pltpu.sync_copy(x_hbm.at[pl.ds(base, CH)], rows)
            c0 = pltpu.async_copy(rows, buf_hbm.at[i0], sem0)
            c1 = pltpu.async_copy(rows, buf_hbm.at[i1], sem1)
            c0.wait()
            c1.wait()
            return acc

        lax.fori_loop(0, TPW // CH, chunk, 0)

    return k(x, slot0, slot1)


def _ffn_body(cnt_ref, buf_ref, w1_ref, b1_ref, w2_ref, b2_ref, y_ref):
    e = pl.program_id(0)
    f = pl.program_id(1)

    @pl.when(jnp.logical_and(e == E, f == 0))
    def _():
        y_ref[...] = jnp.zeros_like(y_ref)

    @pl.when(e < E)
    def _():
        cnt = cnt_ref[jnp.minimum(e, E - 1)]
        for i in range(NCB):
            @pl.when(i * BC < cnt)
            def _():
                rows = pl.ds(i * BC, BC)
                bi = buf_ref[rows, :]
                xl = lax.bitcast_convert_type(bi << 16, jnp.float32)
                xh = lax.bitcast_convert_type(bi & ~0xFFFF, jnp.float32)
                xb = jnp.concatenate([xl, xh], axis=1)
                h = jnp.dot(xb, w1_ref[0],
                            preferred_element_type=jnp.float32)
                h = jnp.maximum(h + b1_ref[0], 0.0)
                part = jnp.dot(h, w2_ref[0], preferred_element_type=jnp.float32)

                @pl.when(f == 0)
                def _():
                    y_ref[rows, :] = part + b2_ref[0]

                @pl.when(f > 0)
                def _():
                    y_ref[rows, :] = y_ref[rows, :] + part


def _ffn(cnt, buf, w1r, b1r, w2r, b2r):
    emax = E - 1
    return pl.pallas_call(
        _ffn_body,
        grid=(E + 1, NFB),
        in_specs=[
            pl.BlockSpec(memory_space=pltpu.SMEM),
            pl.BlockSpec((C, D2), lambda e, f: (e, 0)),
            pl.BlockSpec((1, D, BF), lambda e, f: (jnp.minimum(e, emax), 0, f)),
            pl.BlockSpec((1, 1, BF), lambda e, f: (jnp.minimum(e, emax), 0, f)),
            pl.BlockSpec((1, BF, D), lambda e, f: (jnp.minimum(e, emax), f, 0)),
            pl.BlockSpec((1, 1, D), lambda e, f: (jnp.minimum(e, emax), 0, 0)),
        ],
        out_specs=pl.BlockSpec((C, D), lambda e, f: (e, 0)),
        out_shape=jax.ShapeDtypeStruct((NROWS, D), jnp.float32),
    )(cnt, buf, w1r, b1r, w2r, b2r)


def _combine(y, slot0, slot1, w0, w1):
    mesh = plsc.VectorSubcoreMesh(core_axis_name="c", subcore_axis_name="s")
    s0r = slot0.reshape(NW, NCHC, CHC)
    s1r = slot1.reshape(NW, NCHC, CHC)

    @functools.partial(
        pl.kernel,
        out_type=jax.ShapeDtypeStruct((N, D), jnp.float32),
        mesh=mesh,
        scratch_types=[
            pltpu.VMEM((NCHC, CHC), jnp.int32),
            pltpu.VMEM((NCHC, CHC), jnp.int32),
            pltpu.VMEM((TPW, 16), jnp.float32),
            pltpu.VMEM((TPW, 16), jnp.float32),
            pltpu.VMEM((CHC, D), jnp.float32),
            pltpu.VMEM((CHC, D), jnp.float32),
            pltpu.VMEM((CHC, D), jnp.float32),
            pltpu.VMEM((CHC, D), jnp.float32),
            pltpu.SemaphoreType.DMA,
            pltpu.SemaphoreType.DMA,
            pltpu.SemaphoreType.DMA,
            pltpu.SemaphoreType.DMA,
        ],
    )
    def k(y_hbm, s0_hbm, s1_hbm, w0_hbm, w1_hbm, out_hbm,
          i0, i1, wv0, wv1, r0a, r1a, r0b, r1b, sa0, sa1, sb0, sb1):
        wid = lax.axis_index("s") * 2 + lax.axis_index("c")
        base = wid * TPW
        pltpu.sync_copy(s0_hbm.at[wid], i0)
        pltpu.sync_copy(s1_hbm.at[wid], i1)
        pltpu.sync_copy(w0_hbm.at[pl.ds(base, TPW)], wv0)
        pltpu.sync_copy(w1_hbm.at[pl.ds(base, TPW)], wv1)

        bufs = ((r0a, r1a, sa0, sa1), (r0b, r1b, sb0, sb1))

        def issue(c, bs):
            g0 = pltpu.async_copy(y_hbm.at[i0.at[c]], bs[0], bs[2])
            g1 = pltpu.async_copy(y_hbm.at[i1.at[c]], bs[1], bs[3])
            return g0, g1

        pend = issue(0, bufs[0])
        for c in range(NCHC):
            r0, r1 = bufs[c % 2][0], bufs[c % 2][1]
            pend[0].wait()
            pend[1].wait()
            if c + 1 < NCHC:
                pend = issue(c + 1, bufs[(c + 1) % 2])

            def tok(t, a2):
                b0 = wv0[c * CHC + t, :]
                b1v = wv1[c * CHC + t, :]

                def dslice(j, a3):
                    for u in range(4):
                        sl = pl.ds((j * 4 + u) * 16, 16)
                        r0[t, sl] = b0 * r0[t, sl] + b1v * r1[t, sl]
                    return a3

                return lax.fori_loop(0, D // 64, dslice, a2)

            lax.fori_loop(0, CHC, tok, 0)
            pltpu.sync_copy(r0, out_hbm.at[pl.ds(base + c * CHC, CHC)])

    return k(y, s0r, s1r, w0, w1)


def kernel(hidden_states, Wr, W1, b1, W2, b2):
    orig_shape = hidden_states.shape
    x = hidden_states.reshape(-1, D)

    slot0, slot1, w0, w1, aux, cnt, xpk = _router(x, Wr)
    buf = _dispatch(xpk, slot0, slot1)
    y = _ffn(cnt, buf, W1, b1.reshape(E, 1, F), W2, b2.reshape(E, 1, D))
    out = _combine(y, slot0, slot1, w0, w1)
    return out.reshape(orig_shape), aux


# speedup vs baseline: 1.2709x; 1.0869x over previous
import functools

import jax
import jax.numpy as jnp
from jax import lax
from jax.experimental import pallas as pl
from jax.experimental.pallas import tpu as pltpu
from jax.experimental.pallas import tpu_sc as plsc

E = 8
K = 2
D = 1024
F = 4096
CAP_FACTOR = 1.25

N = 4096
C = int(CAP_FACTOR * N * K / E)
NROWS = (E + 1) * C
DUMMY = E * C

TB = 512
NB = N // TB

NW = 32
TPW = N // NW
CH = 32
CHC = 16
NCHC = TPW // CHC
D2 = D // 2

BC = 256
NCB = C // BC
BF = 2048
NFB = F // BF


def _rne_bf16_bits(v):
    b = lax.bitcast_convert_type(v, jnp.int32)
    return b + 0x7FFF + ((b >> 16) & 1)


def _router_body(x_ref, wr_ref, s0_ref, s1_ref, w0_ref, w1_ref, aux_ref,
                 cnt_ref, xpk_ref, scr):
    i = pl.program_id(0)

    @pl.when(i == 0)
    def _():
        scr[...] = jnp.zeros_like(scr)

    lo = (_rne_bf16_bits(x_ref[:, 0:D2]) >> 16) & 0xFFFF
    hi = _rne_bf16_bits(x_ref[:, D2:D]) & ~0xFFFF
    xpk_ref[...] = lo | hi

    logits = jnp.dot(x_ref[...], wr_ref[...], preferred_element_type=jnp.float32)
    m = jnp.max(logits, axis=-1, keepdims=True)
    ex = jnp.exp(logits - m)
    probs = ex / jnp.sum(ex, axis=-1, keepdims=True)

    lane = lax.broadcasted_iota(jnp.int32, (TB, E), 1)
    m0 = jnp.max(probs, axis=-1, keepdims=True)
    e0 = jnp.min(jnp.where(probs == m0, lane, E), axis=-1, keepdims=True)
    oh0 = lane == e0
    pr2 = jnp.where(oh0, -jnp.inf, probs)
    m1 = jnp.max(pr2, axis=-1, keepdims=True)
    e1 = jnp.min(jnp.where(pr2 == m1, lane, E), axis=-1, keepdims=True)
    oh1 = lane == e1

    ssum = m0 + m1
    g0 = m0 / ssum
    g1 = m1 / ssum

    oh0f = oh0.astype(jnp.float32)
    oh1f = oh1.astype(jnp.float32)
    ohtf = oh0f + oh1f

    ri = lax.broadcasted_iota(jnp.int32, (TB, TB), 0)
    cj = lax.broadcasted_iota(jnp.int32, (TB, TB), 1)
    lstrict = (cj < ri).astype(jnp.float32)
    cume = jnp.dot(lstrict, ohtf, preferred_element_type=jnp.float32)

    carry = scr[0:1, 0:E]
    cnt_excl = cume + carry
    pos0 = jnp.sum(oh0f * cnt_excl, axis=-1, keepdims=True)
    pos1 = jnp.sum(oh1f * cnt_excl, axis=-1, keepdims=True)

    cf = jnp.float32(C)
    keep0 = pos0 < cf
    keep1 = pos1 < cf
    e0f = e0.astype(jnp.float32)
    e1f = e1.astype(jnp.float32)
    slot0 = jnp.where(keep0, e0f * cf + jnp.minimum(pos0, cf - 1), jnp.float32(DUMMY))
    slot1 = jnp.where(keep1, e1f * cf + jnp.minimum(pos1, cf - 1), jnp.float32(DUMMY))

    s0_ref[...] = slot0.astype(jnp.int32)
    s1_ref[...] = slot1.astype(jnp.int32)
    w0_ref[...] = jnp.broadcast_to(jnp.where(keep0, g0, 0.0), (TB, 16))
    w1_ref[...] = jnp.broadcast_to(jnp.where(keep1, g1, 0.0), (TB, 16))

    carry_new = carry + jnp.sum(ohtf, axis=0, keepdims=True)
    me_new = scr[1:2, 0:E] + jnp.sum(probs, axis=0, keepdims=True)
    scr[0:1, 0:E] = carry_new
    scr[1:2, 0:E] = me_new

    @pl.when(i == NB - 1)
    def _():
        ce = carry_new / jnp.float32(K * N)
        me = me_new / jnp.float32(N)
        aux = jnp.float32(E) * jnp.sum(me * ce)
        aux_ref[...] = jnp.full((8, 128), aux, jnp.float32)
        cnt_ref[...] = jnp.broadcast_to(carry_new, (8, E)).astype(jnp.int32)


def _router(x, wr):
    outs = pl.pallas_call(
        _router_body,
        grid=(NB,),
        in_specs=[
            pl.BlockSpec((TB, D), lambda i: (i, 0)),
            pl.BlockSpec((D, E), lambda i: (0, 0)),
        ],
        out_specs=[
            pl.BlockSpec((TB, 1), lambda i: (i, 0)),
            pl.BlockSpec((TB, 1), lambda i: (i, 0)),
            pl.BlockSpec((TB, 16), lambda i: (i, 0)),
            pl.BlockSpec((TB, 16), lambda i: (i, 0)),
            pl.BlockSpec((8, 128), lambda i: (0, 0)),
            pl.BlockSpec((8, E), lambda i: (0, 0)),
            pl.BlockSpec((TB, D2), lambda i: (i, 0)),
        ],
        out_shape=[
            jax.ShapeDtypeStruct((N, 1), jnp.int32),
            jax.ShapeDtypeStruct((N, 1), jnp.int32),
            jax.ShapeDtypeStruct((N, 16), jnp.float32),
            jax.ShapeDtypeStruct((N, 16), jnp.float32),
            jax.ShapeDtypeStruct((8, 128), jnp.float32),
            jax.ShapeDtypeStruct((8, E), jnp.int32),
            jax.ShapeDtypeStruct((N, D2), jnp.int32),
        ],
        scratch_shapes=[pltpu.VMEM((8, 128), jnp.float32)],
    )(x, wr)
    s0, s1, w0, w1, auxb, cnt, xpk = outs
    return s0.reshape(N), s1.reshape(N), w0, w1, auxb[0, 0], cnt[0], xpk


def _dispatch(x, slot0, slot1):
    mesh = plsc.VectorSubcoreMesh(core_axis_name="c", subcore_axis_name="s")

    @functools.partial(
        pl.kernel,
        out_type=jax.ShapeDtypeStruct((NROWS, D2), jnp.int32),
        mesh=mesh,
        scratch_types=[
            pltpu.VMEM((CH,), jnp.int32),
            pltpu.VMEM((CH,), jnp.int32),
            pltpu.VMEM((CH, D2), jnp.int32),
            pltpu.SemaphoreType.DMA,
            pltpu.SemaphoreType.DMA,
        ],
    )
    def k(x_hbm, s0_hbm, s1_hbm, buf_hbm, i0, i1, rows, sem0, sem1):
        wid = lax.axis_index("s") * 2 + lax.axis_index("c")

        def chunk(c, acc):
            base = wid * TPW + c * CH
            pltpu.sync_copy(s0_hbm.at[pl.ds(base, CH)], i0)
            pltpu.sync_copy(s1_hbm.at[pl.ds(base, CH)], i1)
            pltpu.sync_copy(x_hbm.at[pl.ds(base, CH)], rows)
            c0 = pltpu.async_copy(rows, buf_hbm.at[i0], sem0)
            c1 = pltpu.async_copy(rows, buf_hbm.at[i1], sem1)
            c0.wait()
            c1.wait()
            return acc

        lax.fori_loop(0, TPW // CH, chunk, 0)

    return k(x, slot0, slot1)


def _ffn_body(cnt_ref, buf_ref, w1_ref, b1_ref, w2_ref, b2_ref, y_ref):
    e = pl.program_id(0)
    f = pl.program_id(1)

    @pl.when(jnp.logical_and(e == E, f == 0))
    def _():
        y_ref[...] = jnp.zeros_like(y_ref)

    @pl.when(e < E)
    def _():
        cnt = cnt_ref[jnp.minimum(e, E - 1)]
        for i in range(NCB):
            @pl.when(i * BC < cnt)
            def _():
                rows = pl.ds(i * BC, BC)
                bi = buf_ref[rows, :]
                xl = lax.bitcast_convert_type(bi << 16, jnp.float32)
                xh = lax.bitcast_convert_type(bi & ~0xFFFF, jnp.float32)
                xb = jnp.concatenate([xl, xh], axis=1)
                h = jnp.dot(xb, w1_ref[0],
                            preferred_element_type=jnp.float32)
                h = jnp.maximum(h + b1_ref[0], 0.0)
                part = jnp.dot(h, w2_ref[0], preferred_element_type=jnp.float32)

                @pl.when(f == 0)
                def _():
                    y_ref[rows, :] = part + b2_ref[0]

                @pl.when(f > 0)
                def _():
                    y_ref[rows, :] = y_ref[rows, :] + part


def _ffn(cnt, buf, w1r, b1r, w2r, b2r):
    emax = E - 1
    return pl.pallas_call(
        _ffn_body,
        grid=(E + 1, NFB),
        in_specs=[
            pl.BlockSpec(memory_space=pltpu.SMEM),
            pl.BlockSpec((C, D2), lambda e, f: (e, 0)),
            pl.BlockSpec((1, D, BF), lambda e, f: (jnp.minimum(e, emax), 0, f)),
            pl.BlockSpec((1, 1, BF), lambda e, f: (jnp.minimum(e, emax), 0, f)),
            pl.BlockSpec((1, BF, D), lambda e, f: (jnp.minimum(e, emax), f, 0)),
            pl.BlockSpec((1, 1, D), lambda e, f: (jnp.minimum(e, emax), 0, 0)),
        ],
        out_specs=pl.BlockSpec((C, D), lambda e, f: (e, 0)),
        out_shape=jax.ShapeDtypeStruct((NROWS, D), jnp.float32),
    )(cnt, buf, w1r, b1r, w2r, b2r)


def _combine(y, slot0, slot1, w0, w1):
    mesh = plsc.VectorSubcoreMesh(core_axis_name="c", subcore_axis_name="s")
    s0r = slot0.reshape(NW, NCHC, CHC)
    s1r = slot1.reshape(NW, NCHC, CHC)

    @functools.partial(
        pl.kernel,
        out_type=jax.ShapeDtypeStruct((N, D), jnp.float32),
        mesh=mesh,
        scratch_types=[
            pltpu.VMEM((NCHC, CHC), jnp.int32),
            pltpu.VMEM((NCHC, CHC), jnp.int32),
            pltpu.VMEM((TPW, 16), jnp.float32),
            pltpu.VMEM((TPW, 16), jnp.float32),
            pltpu.VMEM((CHC, D), jnp.float32),
            pltpu.VMEM((CHC, D), jnp.float32),
            pltpu.VMEM((CHC, D), jnp.float32),
            pltpu.VMEM((CHC, D), jnp.float32),
            pltpu.SemaphoreType.DMA,
            pltpu.SemaphoreType.DMA,
            pltpu.SemaphoreType.DMA,
            pltpu.SemaphoreType.DMA,
        ],
    )
    def k(y_hbm, s0_hbm, s1_hbm, w0_hbm, w1_hbm, out_hbm,
          i0, i1, wv0, wv1, r0a, r1a, r0b, r1b, sa0, sa1, sb0, sb1):
        wid = lax.axis_index("s") * 2 + lax.axis_index("c")
        base = wid * TPW
        pltpu.sync_copy(s0_hbm.at[wid], i0)
        pltpu.sync_copy(s1_hbm.at[wid], i1)
        pltpu.sync_copy(w0_hbm.at[pl.ds(base, TPW)], wv0)
        pltpu.sync_copy(w1_hbm.at[pl.ds(base, TPW)], wv1)

        bufs = ((r0a, r1a, sa0, sa1), (r0b, r1b, sb0, sb1))

        def issue(c, bs):
            g0 = pltpu.async_copy(y_hbm.at[i0.at[c]], bs[0], bs[2])
            g1 = pltpu.async_copy(y_hbm.at[i1.at[c]], bs[1], bs[3])
            return g0, g1

        pend = issue(0, bufs[0])
        for c in range(NCHC):
            r0, r1 = bufs[c % 2][0], bufs[c % 2][1]
            pend[0].wait()
            pend[1].wait()
            if c + 1 < NCHC:
                pend = issue(c + 1, bufs[(c + 1) % 2])

            def tok(t, a2):
                b0 = wv0[c * CHC + t, :]
                b1v = wv1[c * CHC + t, :]

                def dslice(j, a3):
                    for u in range(4):
                        sl = pl.ds((j * 4 + u) * 16, 16)
                        r0[t, sl] = b0 * r0[t, sl] + b1v * r1[t, sl]
                    return a3

                return lax.fori_loop(0, D // 64, dslice, a2)

            lax.fori_loop(0, CHC, tok, 0)
            pltpu.sync_copy(r0, out_hbm.at[pl.ds(base + c * CHC, CHC)])

    return k(y, s0r, s1r, w0, w1)


def kernel(hidden_states, Wr, W1, b1, W2, b2):
    orig_shape = hidden_states.shape
    x = hidden_states.reshape(-1, D)

    slot0, slot1, w0, w1, aux, cnt, xpk = _router(x, Wr)
    buf = _dispatch(xpk, slot0, slot1)
    y = _ffn(cnt, buf, W1, b1.reshape(E, 1, F), W2, b2.reshape(E, 1, D))
    out = _combine(y, slot0, slot1, w0, w1)
    return out.reshape(orig_shape), aux
